# baseline (speedup 1.0000x reference)
import jax
import jax.numpy as jnp
from jax.experimental import pallas as pl
from jax.experimental.pallas import tpu as pltpu

_BM = 80


def _gconv_kernel(adj_ref, x_ref, w_ref, b_ref, out_ref):
    s = jnp.dot(adj_ref[...], x_ref[...], preferred_element_type=jnp.float32)
    out_ref[...] = (
        jnp.dot(s, w_ref[...], preferred_element_type=jnp.float32) + b_ref[...]
    )


def kernel(x, adj, W, b):
    N, M, Fin = x.shape
    Fout = W.shape[1]
    x2 = x.reshape(N * M, Fin)
    b2 = b.reshape(1, Fout)
    out = pl.pallas_call(
        _gconv_kernel,
        grid=(M // _BM,),
        in_specs=[
            pl.BlockSpec((_BM, M), lambda i: (i, 0)),
            pl.BlockSpec((M, Fin), lambda i: (0, 0)),
            pl.BlockSpec((Fin, Fout), lambda i: (0, 0)),
            pl.BlockSpec((1, Fout), lambda i: (0, 0)),
        ],
        out_specs=pl.BlockSpec((_BM, Fout), lambda i: (i, 0)),
        out_shape=jax.ShapeDtypeStruct((M, Fout), jnp.float32),
        compiler_params=pltpu.CompilerParams(
            dimension_semantics=("parallel",),
        ),
    )(adj, x2, W, b2)
    return out.reshape(N, M, Fout)


# speedup vs baseline: 1.3875x; 1.3875x over previous
import jax
import jax.numpy as jnp
from jax.experimental import pallas as pl
from jax.experimental.pallas import tpu as pltpu

_BM = 400


def _gconv_kernel(adj_ref, x_ref, w_ref, b_ref, out_ref):
    s = jnp.dot(adj_ref[...], x_ref[...], preferred_element_type=jnp.float32)
    out_ref[...] = (
        jnp.dot(s, w_ref[...], preferred_element_type=jnp.float32) + b_ref[...]
    )


def kernel(x, adj, W, b):
    N, M, Fin = x.shape
    Fout = W.shape[1]
    x2 = x.reshape(N * M, Fin)
    b2 = b.reshape(1, Fout)
    out = pl.pallas_call(
        _gconv_kernel,
        grid=(M // _BM,),
        in_specs=[
            pl.BlockSpec((_BM, M), lambda i: (i, 0)),
            pl.BlockSpec((M, Fin), lambda i: (0, 0)),
            pl.BlockSpec((Fin, Fout), lambda i: (0, 0)),
            pl.BlockSpec((1, Fout), lambda i: (0, 0)),
        ],
        out_specs=pl.BlockSpec((_BM, Fout), lambda i: (i, 0)),
        out_shape=jax.ShapeDtypeStruct((M, Fout), jnp.float32),
        compiler_params=pltpu.CompilerParams(
            dimension_semantics=("parallel",),
        ),
    )(adj, x2, W, b2)
    return out.reshape(N, M, Fout)
